# Initial kernel scaffold; baseline (speedup 1.0000x reference)
import jax
import jax.numpy as jnp
from jax.experimental import pallas as pl


def kernel(feature_out, labels, indexes):
    raise NotImplementedError("write your pallas kernel here")


# speedup vs baseline: 3.6144x; 3.6144x over previous
import jax
import jax.numpy as jnp
from jax import lax
from jax.experimental import pallas as pl
from jax.experimental.pallas import tpu as pltpu

B = 4
D = 256
N = 16384
S = 320
NC = 5
TK = 2048
NT = N // TK
UC = 8


def _mx_kernel(idx_ref, mx_ref):
    mx_ref[...] = jnp.max(idx_ref[...], axis=1, keepdims=True)


def _sums_kernel(mx_ref, lab_ref, idx_ref, feat_ref, sums_ref, counts_ref):
    n = pl.program_id(0)
    tt = pl.program_id(1)
    mx = mx_ref[n, 0]
    v = mx * lab_ref[0] + idx_ref[0]
    sidx = lax.broadcasted_iota(jnp.int32, (S, TK), 0)
    onehot = (sidx == v).astype(jnp.float32)
    feat = feat_ref[0]
    part = lax.dot_general(
        onehot, feat,
        dimension_numbers=(((1,), (1,)), ((), ())),
        preferred_element_type=jnp.float32,
    )
    cnt = jnp.sum(onehot, axis=1, keepdims=True)

    @pl.when(tt == 0)
    def _init():
        sums_ref[0] = part
        counts_ref[0] = cnt

    @pl.when(tt != 0)
    def _acc():
        sums_ref[0] += part
        counts_ref[0] += cnt


def _epilogue_kernel(sums_ref, counts_ref, mx_ref, out_ref, mean_s, m_s):
    total = jnp.float32(0.0)
    acc = jnp.float32(0.0)
    for n in range(B):
        cnt = counts_ref[n]
        mean_s[...] = sums_ref[n] / jnp.maximum(cnt, 1.0)
        nonempty = cnt > 0.0
        nseg = jnp.sum(nonempty.astype(jnp.float32))
        vv = lax.broadcasted_iota(jnp.int32, (S, 1), 0).astype(jnp.float32)
        vmax = jnp.max(jnp.where(nonempty, vv, -1.0))
        v2 = jnp.max(jnp.where(nonempty & (vv != vmax), vv, -1.0))
        prev_val = jnp.where(nseg >= 2.0, v2, vmax)
        mxf = mx_ref[n, 0].astype(jnp.float32)
        cls = jnp.ceil(vv / mxf - 1.0)
        last_cls = jnp.ceil(prev_val / mxf - 1.0)
        cls = jnp.where(vv == vmax, last_cls, cls)
        valid = (cnt >= 2.0) & (vv != 0.0) & (nseg > 1.0)
        cidx = lax.broadcasted_iota(jnp.int32, (S, NC), 1).astype(jnp.float32)
        m = (valid & (cls == cidx)).astype(jnp.float32)
        m_s[...] = m
        ks = jnp.sum(m, axis=0, keepdims=True)

        def body(uc, ss):
            chunk = mean_s[pl.ds(uc * UC, UC), :]
            mean = mean_s[...]
            rows = []
            for s in range(UC):
                diff = jnp.abs(mean - chunk[s : s + 1, :])
                rows.append(jnp.sum(diff, axis=1, keepdims=True))
            pd_t = jnp.concatenate(rows, axis=1)
            r = lax.dot_general(
                pd_t, m_s[...],
                dimension_numbers=(((0,), (0,)), ((), ())),
                preferred_element_type=jnp.float32,
            )
            mu = m_s[pl.ds(uc * UC, UC), :]
            return ss + lax.dot_general(
                mu, r,
                dimension_numbers=(((0,), (0,)), ((), ())),
                preferred_element_type=jnp.float32,
            )

        ss = lax.fori_loop(
            0, S // UC, body, jnp.zeros((NC, NC), jnp.float32)
        )

        for i in range(NC - 1):
            for j in range(i + 1, NC):
                npairs = ks[0, i] * ks[0, j]
                denom = jnp.maximum(npairs, 1.0) * jnp.float32(D)
                ret = ss[i, j] / denom
                ret = jnp.where(ret < 1.0, 0.5 * ret * ret, ret - 0.5)
                flag = (npairs > 0.0).astype(jnp.float32)
                total += flag
                acc += ret * flag

    mean_loss = acc / jnp.maximum(total, 1.0)
    loss = jnp.where(total > 0.0, -mean_loss, 0.0)
    loss = jnp.where(loss == 0.0, -jnp.float32(B), loss)
    out_ref[0, 0] = -jnp.log(-loss / jnp.float32(B))


def kernel(feature_out, labels, indexes):
    feat = feature_out.reshape(B, D, N)
    lab = labels.reshape(B, 1, N).astype(jnp.int32)
    idx = indexes.reshape(B, 1, N).astype(jnp.int32)
    idx2 = indexes.reshape(B, N).astype(jnp.int32)

    mx = pl.pallas_call(
        _mx_kernel,
        out_shape=jax.ShapeDtypeStruct((B, 1), jnp.int32),
    )(idx2)

    sums, counts = pl.pallas_call(
        _sums_kernel,
        grid=(B, NT),
        in_specs=[
            pl.BlockSpec(memory_space=pltpu.SMEM),
            pl.BlockSpec((1, 1, TK), lambda n, t: (n, 0, t)),
            pl.BlockSpec((1, 1, TK), lambda n, t: (n, 0, t)),
            pl.BlockSpec((1, D, TK), lambda n, t: (n, 0, t)),
        ],
        out_specs=[
            pl.BlockSpec((1, S, D), lambda n, t: (n, 0, 0)),
            pl.BlockSpec((1, S, 1), lambda n, t: (n, 0, 0)),
        ],
        out_shape=[
            jax.ShapeDtypeStruct((B, S, D), jnp.float32),
            jax.ShapeDtypeStruct((B, S, 1), jnp.float32),
        ],
    )(mx, lab, idx, feat)

    out = pl.pallas_call(
        _epilogue_kernel,
        in_specs=[
            pl.BlockSpec(memory_space=pltpu.VMEM),
            pl.BlockSpec(memory_space=pltpu.VMEM),
            pl.BlockSpec(memory_space=pltpu.SMEM),
        ],
        out_specs=pl.BlockSpec(memory_space=pltpu.SMEM),
        out_shape=jax.ShapeDtypeStruct((1, 1), jnp.float32),
        scratch_shapes=[
            pltpu.VMEM((S, D), jnp.float32),
            pltpu.VMEM((S, NC), jnp.float32),
        ],
    )(sums, counts, mx)
    return out.reshape(1)
